# baseline (speedup 1.0000x reference)
import functools

import jax
import jax.numpy as jnp
from jax import lax
from jax.experimental import pallas as pl
from jax.experimental.pallas import tpu as pltpu
from jax.experimental.pallas import tpu_sc as plsc

_NC = 2
_NS = 16
_NW = _NC * _NS
_L = 16
_CH = 128


def _sc_body(bpw, d, k, z_hbm, a_hbm, out_hbm, zbuf, abuf, obuf, zsem, asem, osem):
    wid = lax.axis_index("s") * _NC + lax.axis_index("c")
    base = wid * bpw
    kf = jnp.float32(k)
    kmax = jnp.int32(k - 1)
    nch = bpw // _CH

    def zcopy(ci, b):
        return pltpu.make_async_copy(
            z_hbm.at[pl.ds(base + ci * _CH, _CH), :],
            zbuf.at[pl.ds(b * _CH, _CH), :],
            zsem,
        )

    def acopy(ci, b):
        return pltpu.make_async_copy(
            a_hbm.at[pl.ds(base + ci * _CH, _CH), :],
            abuf.at[pl.ds(b * _CH, _CH), :],
            asem,
        )

    def odrain(b):
        return pltpu.make_async_copy(
            obuf.at[pl.ds(b * _CH, _CH)], out_hbm.at[pl.ds(base, _CH)], osem
        )

    zcopy(0, 0).start()
    acopy(0, 0).start()

    def chunk_work(ci, b):
        cbase = base + ci * _CH

        @pl.when(ci >= 2)
        def _():
            odrain(b).wait()

        zcopy(ci, b).wait()
        acopy(ci, b).wait()

        @pl.when(ci + 1 < nch)
        def _():
            zcopy(ci + 1, 1 - b).start()
            acopy(ci + 1, 1 - b).start()

        zeros = jnp.zeros((_L,), jnp.int32)

        def step(v, _):
            r0 = b * _CH + v * _L
            rows = lax.iota(jnp.int32, _L) + r0
            zc = plsc.load_gather(zbuf, [rows, zeros])
            idx = jnp.clip((zc * kf).astype(jnp.int32), 0, kmax)
            picked = plsc.load_gather(abuf, [rows, idx])
            obuf[pl.ds(r0, _L)] = picked * 0.999
            return 0

        lax.fori_loop(0, _CH // _L, step, 0)
        pltpu.make_async_copy(
            obuf.at[pl.ds(b * _CH, _CH)], out_hbm.at[pl.ds(cbase, _CH)], osem
        ).start()

    def loop_body(i, _):
        chunk_work(2 * i, 0)
        chunk_work(2 * i + 1, 1)
        return 0

    lax.fori_loop(0, nch // 2, loop_body, 0)
    odrain(0).wait()
    odrain(1).wait()


def kernel(z, a):
    b, d = z.shape
    _, k = a.shape
    bpw = b // _NW
    assert b % (_NW * _CH * 2) == 0
    mesh = plsc.VectorSubcoreMesh(
        core_axis_name="c", subcore_axis_name="s", num_cores=_NC, num_subcores=_NS
    )
    fn = pl.kernel(
        functools.partial(_sc_body, bpw, d, k),
        out_type=jax.ShapeDtypeStruct((b,), jnp.float32),
        mesh=mesh,
        compiler_params=pltpu.CompilerParams(
            needs_layout_passes=False, use_tc_tiling_on_sc=True
        ),
        scratch_types=[
            pltpu.VMEM((2 * _CH, d), jnp.float32),
            pltpu.VMEM((2 * _CH, k), jnp.float32),
            pltpu.VMEM((2 * _CH,), jnp.float32),
            pltpu.SemaphoreType.DMA,
            pltpu.SemaphoreType.DMA,
            pltpu.SemaphoreType.DMA,
        ],
    )
    return fn(z, a)


# speedup vs baseline: 1.1001x; 1.1001x over previous
import functools

import jax
import jax.numpy as jnp
from jax import lax
from jax.experimental import pallas as pl
from jax.experimental.pallas import tpu as pltpu
from jax.experimental.pallas import tpu_sc as plsc

_NC = 2
_NS = 16
_NW = _NC * _NS
_L = 16
_CH = 64
_NB = 4
_OB = 2048
_OR = 2


def _sc_body(bpw, d, k, z_hbm, a_hbm, out_hbm, zbuf, abuf, obuf, zsem, asem, osem):
    wid = lax.axis_index("s") * _NC + lax.axis_index("c")
    base = wid * bpw
    kf = jnp.float32(k)
    kmax = jnp.int32(k - 1)
    nch = bpw // _CH
    chunks_per_ob = _OB // _CH

    def zcopy(ci, b):
        return pltpu.make_async_copy(
            z_hbm.at[pl.ds(base + ci * _CH, _CH), :],
            zbuf.at[pl.ds(b * _CH, _CH), :],
            zsem,
        )

    def acopy(ci, b):
        return pltpu.make_async_copy(
            a_hbm.at[pl.ds(base + ci * _CH, _CH), :],
            abuf.at[pl.ds(b * _CH, _CH), :],
            asem,
        )

    def ocopy(obi, ob):
        return pltpu.make_async_copy(
            obuf.at[pl.ds(ob * _OB, _OB)],
            out_hbm.at[pl.ds(base + obi * _OB, _OB)],
            osem,
        )

    for b in range(_NB - 1):
        zcopy(b, b).start()
        acopy(b, b).start()

    zeros = jnp.zeros((_L,), jnp.int32)

    def chunk_work(ci, b):
        zcopy(ci, b).wait()
        acopy(ci, b).wait()

        @pl.when(ci + _NB - 1 < nch)
        def _():
            zcopy(ci + _NB - 1, (b + _NB - 1) % _NB).start()
            acopy(ci + _NB - 1, (b + _NB - 1) % _NB).start()

        oslot = (ci // chunks_per_ob) % _OR

        for v in range(_CH // _L):
            r0 = b * _CH + v * _L
            rows = lax.iota(jnp.int32, _L) + r0
            zc = plsc.load_gather(zbuf, [rows, zeros])
            idx = jnp.clip((zc * kf).astype(jnp.int32), 0, kmax)
            picked = plsc.load_gather(abuf, [rows, idx])
            obuf[pl.ds(oslot * _OB + (ci % chunks_per_ob) * _CH + v * _L, _L)] = (
                picked * 0.999
            )

        @pl.when((ci + 1) % chunks_per_ob == 0)
        def _():
            ocopy(ci // chunks_per_ob, oslot).start()

        @pl.when((ci % chunks_per_ob == 0) & (ci // chunks_per_ob >= _OR))
        def _():
            pltpu.make_async_copy(
                obuf.at[pl.ds(oslot * _OB, _OB)],
                out_hbm.at[pl.ds(base, _OB)],
                osem,
            ).wait()

        return 0

    def loop_body(i, _):
        for s in range(_NB):
            ci = i * _NB + s
            chunk_work(ci, s)
        return 0

    lax.fori_loop(0, nch // _NB, loop_body, 0)
    for _ in range(_OR):
        pltpu.make_async_copy(
            obuf.at[pl.ds(0, _OB)], out_hbm.at[pl.ds(base, _OB)], osem
        ).wait()


def kernel(z, a):
    b, d = z.shape
    _, k = a.shape
    bpw = b // _NW
    assert b % (_NW * _CH * _NB) == 0
    assert _OB % (_CH * _NB) == 0
    mesh = plsc.VectorSubcoreMesh(
        core_axis_name="c", subcore_axis_name="s", num_cores=_NC, num_subcores=_NS
    )
    fn = pl.kernel(
        functools.partial(_sc_body, bpw, d, k),
        out_type=jax.ShapeDtypeStruct((b,), jnp.float32),
        mesh=mesh,
        compiler_params=pltpu.CompilerParams(
            needs_layout_passes=False, use_tc_tiling_on_sc=True
        ),
        scratch_types=[
            pltpu.VMEM((_NB * _CH, d), jnp.float32),
            pltpu.VMEM((_NB * _CH, k), jnp.float32),
            pltpu.VMEM((_OR * _OB,), jnp.float32),
            pltpu.SemaphoreType.DMA,
            pltpu.SemaphoreType.DMA,
            pltpu.SemaphoreType.DMA,
        ],
    )
    return fn(z, a)


# speedup vs baseline: 2.0377x; 1.8522x over previous
import functools

import jax
import jax.numpy as jnp
from jax import lax
from jax.experimental import pallas as pl
from jax.experimental.pallas import tpu as pltpu
from jax.experimental.pallas import tpu_sc as plsc

_NC = 2
_NS = 16
_NW = _NC * _NS
_L = 16
_CH = 2048
_GW = 128
_NG = _CH // _GW


def _sc_body(bpw, k, zc_hbm, a_hbm, out_hbm, zbuf, idxbuf, gbuf, zsem, gsem, osem):
    wid = lax.axis_index("s") * _NC + lax.axis_index("c")
    base = wid * bpw
    kf = jnp.float32(k)
    kmax = jnp.int32(k - 1)
    nch = bpw // _CH

    def zcopy(ci, b):
        return pltpu.make_async_copy(
            zc_hbm.at[pl.ds(base + ci * _CH, _CH)],
            zbuf.at[pl.ds(b * _CH, _CH)],
            zsem,
        )

    def gdrain(b):
        return pltpu.make_async_copy(
            a_hbm.at[idxbuf.at[b, 0]], gbuf.at[b, pl.ds(0, _GW)], gsem
        )

    def odrain(b):
        return pltpu.make_async_copy(
            gbuf.at[b], out_hbm.at[pl.ds(base, _CH)], osem
        )

    zcopy(0, 0).start()

    def chunk_work(ci, b):
        cbase = base + ci * _CH
        zcopy(ci, b).wait()

        @pl.when(ci + 1 < nch)
        def _():
            zcopy(ci + 1, 1 - b).start()

        @pl.when(ci >= 2)
        def _():
            odrain(b).wait()

        def idx_step(g, _):
            for l in range(_GW // _L):
                r0 = g * _GW + l * _L
                zc = zbuf[pl.ds(b * _CH + r0, _L)]
                idx = jnp.clip((zc * kf).astype(jnp.int32), 0, kmax)
                flat = (cbase + r0 + lax.iota(jnp.int32, _L)) * k + idx
                idxbuf[b, g, pl.ds(l * _L, _L)] = flat
            pltpu.make_async_copy(
                a_hbm.at[idxbuf.at[b, g]], gbuf.at[b, pl.ds(g * _GW, _GW)], gsem
            ).start()
            return 0

        lax.fori_loop(0, _NG, idx_step, 0)

        def drain_step(g, _):
            gdrain(b).wait()
            return 0

        lax.fori_loop(0, _NG, drain_step, 0)

        def scale_step(g, _):
            for l in range(_GW // _L):
                o = g * _GW + l * _L
                gbuf[b, pl.ds(o, _L)] = gbuf[b, pl.ds(o, _L)] * 0.999
            return 0

        lax.fori_loop(0, _NG, scale_step, 0)
        pltpu.make_async_copy(
            gbuf.at[b], out_hbm.at[pl.ds(cbase, _CH)], osem
        ).start()

    def loop_body(i, _):
        chunk_work(2 * i, 0)
        chunk_work(2 * i + 1, 1)
        return 0

    lax.fori_loop(0, nch // 2, loop_body, 0)
    odrain(0).wait()
    odrain(1).wait()


def kernel(z, a):
    b, d = z.shape
    _, k = a.shape
    bpw = b // _NW
    assert b % (_NW * _CH * 2) == 0
    zc = z[:, 0]
    a_flat = a.reshape(-1)
    mesh = plsc.VectorSubcoreMesh(
        core_axis_name="c", subcore_axis_name="s", num_cores=_NC, num_subcores=_NS
    )
    fn = pl.kernel(
        functools.partial(_sc_body, bpw, k),
        out_type=jax.ShapeDtypeStruct((b,), jnp.float32),
        mesh=mesh,
        compiler_params=pltpu.CompilerParams(needs_layout_passes=False),
        scratch_types=[
            pltpu.VMEM((2 * _CH,), jnp.float32),
            pltpu.VMEM((2, _NG, _GW), jnp.int32),
            pltpu.VMEM((2, _CH), jnp.float32),
            pltpu.SemaphoreType.DMA,
            pltpu.SemaphoreType.DMA,
            pltpu.SemaphoreType.DMA,
        ],
    )
    return fn(zc, a_flat)
